# Initial kernel scaffold; baseline (speedup 1.0000x reference)
import jax
import jax.numpy as jnp
from jax.experimental import pallas as pl


def kernel(x, tables, W1, b1, W2, b2):
    raise NotImplementedError("write your pallas kernel here")


# speedup vs baseline: 7.9518x; 7.9518x over previous
import functools

import jax
import jax.numpy as jnp
from jax import lax
from jax.experimental import pallas as pl
from jax.experimental.pallas import tpu as pltpu
from jax.experimental.pallas import tpu_sc as plsc

B = 16384
F = 26
V = 100000
D = 16
H1 = 128
H2 = 64

NW = 32
R = B * F
RPW = R // NW
GW = 128
G = 8
ROWS_PER_GROUP = G * GW
NGROUPS = RPW // ROWS_PER_GROUP
K = RPW // GW


def _sc_gather(tab_flat, idx3):
  mesh = plsc.VectorSubcoreMesh(core_axis_name="c", subcore_axis_name="s")

  @functools.partial(
      pl.kernel,
      out_type=jax.ShapeDtypeStruct((R, D), jnp.float32),
      mesh=mesh,
      compiler_params=pltpu.CompilerParams(use_tc_tiling_on_sc=False),
      scratch_types=[
          pltpu.VMEM((K, GW), jnp.int32),
          pltpu.VMEM((ROWS_PER_GROUP, D), jnp.float32),
          pltpu.SemaphoreType.DMA,
      ],
  )
  def body(tab_hbm, idx_hbm, out_hbm, idx_v, rows_v, sem):
    num_s = lax.axis_size("s")
    wid = lax.axis_index("c") * num_s + lax.axis_index("s")
    base = wid * RPW
    pltpu.sync_copy(idx_hbm.at[wid], idx_v)

    def group(g, carry):
      copies = []
      for j in range(G):
        c = pltpu.async_copy(
            tab_hbm.at[idx_v.at[g * G + j]],
            rows_v.at[pl.ds(j * GW, GW)],
            sem,
        )
        copies.append(c)
      for c in copies:
        c.wait()
      pltpu.sync_copy(
          rows_v, out_hbm.at[pl.ds(base + g * ROWS_PER_GROUP, ROWS_PER_GROUP)]
      )
      return carry

    lax.fori_loop(0, NGROUPS, group, 0)

  return body(tab_flat, idx3)


def _mlp(x_cat, W1, b1, W2, b2):
  BB = 2048

  def body(x_ref, w1_ref, b1_ref, w2_ref, b2_ref, o_ref):
    h = jnp.dot(x_ref[...], w1_ref[...], preferred_element_type=jnp.float32)
    h = h + b1_ref[...]
    h = jnp.where(h >= 0, h, 0.01 * h)
    h = jnp.dot(h, w2_ref[...], preferred_element_type=jnp.float32)
    h = h + b2_ref[...]
    o_ref[...] = jnp.where(h >= 0, h, 0.01 * h)

  return pl.pallas_call(
      body,
      grid=(B // BB,),
      in_specs=[
          pl.BlockSpec((BB, F * D), lambda i: (i, 0)),
          pl.BlockSpec((F * D, H1), lambda i: (0, 0)),
          pl.BlockSpec((1, H1), lambda i: (0, 0)),
          pl.BlockSpec((H1, H2), lambda i: (0, 0)),
          pl.BlockSpec((1, H2), lambda i: (0, 0)),
      ],
      out_specs=pl.BlockSpec((BB, H2), lambda i: (i, 0)),
      out_shape=jax.ShapeDtypeStruct((B, H2), jnp.float32),
  )(x_cat, W1, b1.reshape(1, H1), W2, b2.reshape(1, H2))


def kernel(x, tables, W1, b1, W2, b2):
  x = x.astype(jnp.int32)
  offs = (jnp.arange(F, dtype=jnp.int32) * V)[None, :]
  idx3 = (x + offs).reshape(NW, K, GW)
  tab_flat = tables.reshape(F * V, D)
  emb = _sc_gather(tab_flat, idx3)
  x_cat = emb.reshape(B, F * D)
  return _mlp(x_cat, W1, b1, W2, b2)


# speedup vs baseline: 8.0260x; 1.0093x over previous
import functools

import jax
import jax.numpy as jnp
from jax import lax
from jax.experimental import pallas as pl
from jax.experimental.pallas import tpu as pltpu
from jax.experimental.pallas import tpu_sc as plsc

B = 16384
F = 26
V = 100000
D = 16
H1 = 128
H2 = 64

NW = 32
R = B * F
RPW = R // NW
NCH = 8
CH = RPW // NCH


def _sc_gather(tab_flat, idx3):
  mesh = plsc.VectorSubcoreMesh(core_axis_name="c", subcore_axis_name="s")

  @functools.partial(
      pl.kernel,
      out_type=jax.ShapeDtypeStruct((R, D), jnp.float32),
      mesh=mesh,
      compiler_params=pltpu.CompilerParams(use_tc_tiling_on_sc=False),
      scratch_types=[
          pltpu.VMEM((NCH, CH), jnp.int32),
          pltpu.VMEM((2, CH, D), jnp.float32),
          pltpu.SemaphoreType.DMA,
          pltpu.SemaphoreType.DMA,
          pltpu.SemaphoreType.DMA,
          pltpu.SemaphoreType.DMA,
      ],
  )
  def body(tab_hbm, idx_hbm, out_hbm, idx_v, rows_v, g0, g1, o0, o1):
    num_s = lax.axis_size("s")
    wid = lax.axis_index("c") * num_s + lax.axis_index("s")
    base = wid * RPW
    gsem = (g0, g1)
    osem = (o0, o1)
    pltpu.sync_copy(idx_hbm.at[wid], idx_v)

    gathers = [None] * NCH
    outs = [None] * NCH
    gathers[0] = pltpu.async_copy(tab_hbm.at[idx_v.at[0]], rows_v.at[0],
                                  gsem[0])
    for i in range(NCH):
      p = i % 2
      if i + 1 < NCH:
        if i >= 1:
          outs[i - 1].wait()
        gathers[i + 1] = pltpu.async_copy(
            tab_hbm.at[idx_v.at[i + 1]], rows_v.at[1 - p], gsem[1 - p])
      gathers[i].wait()
      outs[i] = pltpu.async_copy(
          rows_v.at[p], out_hbm.at[pl.ds(base + i * CH, CH)], osem[p])
    outs[NCH - 2].wait()
    outs[NCH - 1].wait()

  return body(tab_flat, idx3)


def _mlp(x_cat, W1, b1, W2, b2):
  BB = 2048

  def body(x_ref, w1_ref, b1_ref, w2_ref, b2_ref, o_ref):
    h = jnp.dot(x_ref[...], w1_ref[...], preferred_element_type=jnp.float32)
    h = h + b1_ref[...]
    h = jnp.where(h >= 0, h, 0.01 * h)
    h = jnp.dot(h, w2_ref[...], preferred_element_type=jnp.float32)
    h = h + b2_ref[...]
    o_ref[...] = jnp.where(h >= 0, h, 0.01 * h)

  return pl.pallas_call(
      body,
      grid=(B // BB,),
      in_specs=[
          pl.BlockSpec((BB, F * D), lambda i: (i, 0)),
          pl.BlockSpec((F * D, H1), lambda i: (0, 0)),
          pl.BlockSpec((1, H1), lambda i: (0, 0)),
          pl.BlockSpec((H1, H2), lambda i: (0, 0)),
          pl.BlockSpec((1, H2), lambda i: (0, 0)),
      ],
      out_specs=pl.BlockSpec((BB, H2), lambda i: (i, 0)),
      out_shape=jax.ShapeDtypeStruct((B, H2), jnp.float32),
  )(x_cat, W1, b1.reshape(1, H1), W2, b2.reshape(1, H2))


def kernel(x, tables, W1, b1, W2, b2):
  x = x.astype(jnp.int32)
  offs = (jnp.arange(F, dtype=jnp.int32) * V)[None, :]
  idx3 = (x + offs).reshape(NW, NCH, CH)
  tab_flat = tables.reshape(F * V, D)
  emb = _sc_gather(tab_flat, idx3)
  x_cat = emb.reshape(B, F * D)
  return _mlp(x_cat, W1, b1, W2, b2)
